# Initial kernel scaffold; baseline (speedup 1.0000x reference)
import jax
import jax.numpy as jnp
from jax.experimental import pallas as pl


def kernel(cache, s, order):
    raise NotImplementedError("write your pallas kernel here")


# speedup vs baseline: 1.6272x; 1.6272x over previous
import jax
import jax.numpy as jnp
from jax import lax
from jax.experimental import pallas as pl
from jax.experimental.pallas import tpu as pltpu
from jax.experimental.pallas import tpu_sc as plsc

B, T, D = 32, 2047, 1024
NC, NS = 2, 16
CHUNK = 40
SC_ROWS = 2040
NFULL = SC_ROWS // CHUNK


def _sc_body(cache_hbm, s_hbm, order_hbm, out_hbm, ord_v, buf0, buf1, gsem0, gsem1, osem0, osem1):
    wid = lax.axis_index("s") * NC + lax.axis_index("c")

    pltpu.sync_copy(order_hbm, ord_v)
    lanes = lax.iota(jnp.int32, 16)
    zero = jnp.zeros((16,), jnp.int32)
    va = jnp.where(lanes == wid, ord_v[pl.ds(0, 16)], zero)
    vb = jnp.where(lanes + 16 == wid, ord_v[pl.ds(16, 16)], zero)
    src = jnp.max(va + vb)

    bufs = (buf0, buf1)
    gsems = (gsem0, gsem1)
    osems = (osem0, osem1)
    gathers = [None] * NFULL
    stores = [None] * NFULL

    def gather(c):
        return pltpu.async_copy(
            cache_hbm.at[src, pl.ds(c * CHUNK, CHUNK)], bufs[c % 2], gsems[c % 2]
        )

    gathers[0] = gather(0)
    for c in range(NFULL):
        p = c % 2
        if c + 1 < NFULL:
            if c >= 1:
                stores[c - 1].wait()
            gathers[c + 1] = gather(c + 1)
        gathers[c].wait()
        stores[c] = pltpu.async_copy(
            bufs[p], out_hbm.at[wid, pl.ds(c * CHUNK, CHUNK)], osems[p]
        )
    stores[NFULL - 2].wait()
    stores[NFULL - 1].wait()


def _sc_bulk(cache, s, order):
    mesh = plsc.VectorSubcoreMesh(
        core_axis_name="c", subcore_axis_name="s", num_cores=NC, num_subcores=NS
    )
    return pl.kernel(
        _sc_body,
        out_type=jax.ShapeDtypeStruct((B, T + 1, D), jnp.float32),
        mesh=mesh,
        compiler_params=pltpu.CompilerParams(needs_layout_passes=False),
        scratch_types=[
            pltpu.VMEM((B,), jnp.int32),
            pltpu.VMEM((CHUNK, D), jnp.float32),
            pltpu.VMEM((CHUNK, D), jnp.float32),
            pltpu.SemaphoreType.DMA,
            pltpu.SemaphoreType.DMA,
            pltpu.SemaphoreType.DMA,
            pltpu.SemaphoreType.DMA,
        ],
    )(cache, s, order)


def _tc_tail_body(order_ref, cache_ref, s_ref, prev_ref, out_ref):
    del order_ref, prev_ref
    blk = cache_ref[0]
    out_ref[0, :7] = blk[:7]
    out_ref[0, 7:8] = s_ref[0]


def _tc_tail(cache, s, order, prev):
    grid_spec = pltpu.PrefetchScalarGridSpec(
        num_scalar_prefetch=1,
        grid=(B,),
        in_specs=[
            pl.BlockSpec((1, 8, D), lambda i, ord_ref: (ord_ref[i], T // 8, 0)),
            pl.BlockSpec((1, 1, D), lambda i, ord_ref: (ord_ref[i], 0, 0)),
            pl.BlockSpec(memory_space=pl.ANY),
        ],
        out_specs=pl.BlockSpec((1, 8, D), lambda i, ord_ref: (i, T // 8, 0)),
    )
    return pl.pallas_call(
        _tc_tail_body,
        grid_spec=grid_spec,
        out_shape=jax.ShapeDtypeStruct((B, T + 1, D), jnp.float32),
        input_output_aliases={3: 0},
    )(order, cache, s, prev)


@jax.jit
def kernel(cache, s, order):
    return _tc_tail(cache, s, order, _sc_bulk(cache, s, order))
